# baseline (speedup 1.0000x reference)
import jax
import jax.numpy as jnp
import numpy as np
from jax.experimental import pallas as pl

_N = 5000
_T = 1024
_NPAD = 5120
_NT = _NPAD // _T
_W = 1024
_NC = _NPAD // _W
_THR = float(np.float32(0.7))
_EPS = float(np.float32(1e-8))


def _overlap(ax1, ay1, ax2, ay2, aarea, bx1, by1, bx2, by2, barea):
    ix1 = jnp.maximum(ax1, bx1)
    iy1 = jnp.maximum(ay1, by1)
    ix2 = jnp.minimum(ax2, bx2)
    iy2 = jnp.minimum(ay2, by2)
    iw = jnp.maximum(ix2 - ix1, jnp.float32(0.0))
    ih = jnp.maximum(iy2 - iy1, jnp.float32(0.0))
    inter = iw * ih
    union = (aarea + barea) - inter
    iou = inter / (union + jnp.float32(_EPS))
    return iou > jnp.float32(_THR)


def _nms_body(cols_ref, keep_ref):
    keep_ref[...] = jnp.ones((1, _NPAD), jnp.float32)

    def tile_step(j, carry):
        off = j * _T
        tc = jnp.transpose(cols_ref[0:8, pl.ds(off, _T)])
        tx1 = tc[:, 0:1]
        ty1 = tc[:, 1:2]
        tx2 = tc[:, 2:3]
        ty2 = tc[:, 3:4]
        tar = tc[:, 4:5]
        sx1 = cols_ref[0:1, pl.ds(off, _T)]
        sy1 = cols_ref[1:2, pl.ds(off, _T)]
        sx2 = cols_ref[2:3, pl.ds(off, _T)]
        sy2 = cols_ref[3:4, pl.ds(off, _T)]
        sar = cols_ref[4:5, pl.ds(off, _T)]

        over = _overlap(tx1, ty1, tx2, ty2, tar, sx1, sy1, sx2, sy2, sar)
        ra = jax.lax.broadcasted_iota(jnp.int32, (_T, _T), 0)
        rb = jax.lax.broadcasted_iota(jnp.int32, (_T, _T), 1)
        sf = jnp.where(over & (ra < rb), jnp.float32(1.0), jnp.float32(0.0))

        alive0 = keep_ref[0:1, pl.ds(off, _T)]

        def fp_cond(c):
            return c[1]

        def fp_body(c):
            cum, _ = c
            s1 = jnp.dot(cum, sf, preferred_element_type=jnp.float32)
            cso = jnp.where(s1 == 0, cum, jnp.float32(0.0))
            s2 = jnp.dot(cso, sf, preferred_element_type=jnp.float32)
            nxt = jnp.where(s2 == 0, cum, jnp.float32(0.0))
            return nxt, jnp.any(nxt != cum)

        cum, _ = jax.lax.while_loop(
            fp_cond, fp_body, (alive0, jnp.bool_(True)))
        keep_ref[0:1, pl.ds(off, _T)] = cum

        def chunk_step(k, c2):
            cb = k * _W
            bx1 = cols_ref[0:1, pl.ds(cb, _W)]
            by1 = cols_ref[1:2, pl.ds(cb, _W)]
            bx2 = cols_ref[2:3, pl.ds(cb, _W)]
            by2 = cols_ref[3:4, pl.ds(cb, _W)]
            bar = cols_ref[4:5, pl.ds(cb, _W)]
            oc = _overlap(tx1, ty1, tx2, ty2, tar, bx1, by1, bx2, by2, bar)
            scf = jnp.where(oc, jnp.float32(1.0), jnp.float32(0.0))
            hits = jnp.dot(cum, scf, preferred_element_type=jnp.float32)
            colid = cb + jax.lax.broadcasted_iota(jnp.int32, (1, _W), 1)
            kill = (hits > 0) & (colid >= off + _T)
            cur = keep_ref[0:1, pl.ds(cb, _W)]
            keep_ref[0:1, pl.ds(cb, _W)] = jnp.where(
                kill, jnp.float32(0.0), cur)
            return c2

        kmin = (off + _T) // _W
        jax.lax.fori_loop(kmin, _NC, chunk_step, 0)
        return carry

    jax.lax.fori_loop(0, _NT, tile_step, 0)
    keep_ref[...] = keep_ref[...] * cols_ref[5:6, :]


def _run_nms(cols, interpret=False):
    return pl.pallas_call(
        _nms_body,
        out_shape=jax.ShapeDtypeStruct((1, _NPAD), jnp.float32),
        interpret=interpret,
    )(cols)


def kernel(boxes, scores):
    n = boxes.shape[0]
    idx = jnp.arange(n, dtype=jnp.int32)
    sneg, sx1, sy1, sx2, sy2, sidx = jax.lax.sort(
        (-scores, boxes[:, 0], boxes[:, 1], boxes[:, 2], boxes[:, 3], idx),
        num_keys=1, is_stable=True)
    ssc = -sneg
    area = jnp.maximum(sx2 - sx1, 0.0) * jnp.maximum(sy2 - sy1, 0.0)
    feats = jnp.stack([sx1, sy1, sx2, sy2, area, ssc], axis=0)
    cols = jnp.zeros((8, _NPAD), jnp.float32).at[:6, :n].set(feats)
    masked_sorted = _run_nms(cols)[0, :n]
    _, out = jax.lax.sort((sidx, masked_sorted), num_keys=1)
    return out


# speedup vs baseline: 1.0522x; 1.0522x over previous
import jax
import jax.numpy as jnp
import numpy as np
from jax.experimental import pallas as pl

_N = 5000
_T = 1280
_NPAD = 5120
_NT = _NPAD // _T
_W = 1280
_NC = _NPAD // _W
_THR = float(np.float32(0.7))
_EPS = float(np.float32(1e-8))


def _overlap(ax1, ay1, ax2, ay2, aarea, bx1, by1, bx2, by2, barea):
    ix1 = jnp.maximum(ax1, bx1)
    iy1 = jnp.maximum(ay1, by1)
    ix2 = jnp.minimum(ax2, bx2)
    iy2 = jnp.minimum(ay2, by2)
    iw = jnp.maximum(ix2 - ix1, jnp.float32(0.0))
    ih = jnp.maximum(iy2 - iy1, jnp.float32(0.0))
    inter = iw * ih
    union = (aarea + barea) - inter
    iou = inter / (union + jnp.float32(_EPS))
    return iou > jnp.float32(_THR)


def _nms_body(cols_ref, keep_ref):
    keep_ref[...] = jnp.ones((1, _NPAD), jnp.float32)

    def tile_step(j, carry):
        off = j * _T
        tc = jnp.transpose(cols_ref[0:8, pl.ds(off, _T)])
        tx1 = tc[:, 0:1]
        ty1 = tc[:, 1:2]
        tx2 = tc[:, 2:3]
        ty2 = tc[:, 3:4]
        tar = tc[:, 4:5]
        sx1 = cols_ref[0:1, pl.ds(off, _T)]
        sy1 = cols_ref[1:2, pl.ds(off, _T)]
        sx2 = cols_ref[2:3, pl.ds(off, _T)]
        sy2 = cols_ref[3:4, pl.ds(off, _T)]
        sar = cols_ref[4:5, pl.ds(off, _T)]

        over = _overlap(tx1, ty1, tx2, ty2, tar, sx1, sy1, sx2, sy2, sar)
        ra = jax.lax.broadcasted_iota(jnp.int32, (_T, _T), 0)
        rb = jax.lax.broadcasted_iota(jnp.int32, (_T, _T), 1)
        sf = jnp.where(over & (ra < rb), jnp.float32(1.0), jnp.float32(0.0))

        alive0 = keep_ref[0:1, pl.ds(off, _T)]

        def fp_cond(c):
            return c[1]

        def fp_body(c):
            cum, _ = c
            s1 = jnp.dot(cum, sf, preferred_element_type=jnp.float32)
            cso = jnp.where(s1 == 0, cum, jnp.float32(0.0))
            s2 = jnp.dot(cso, sf, preferred_element_type=jnp.float32)
            nxt = jnp.where(s2 == 0, cum, jnp.float32(0.0))
            return nxt, jnp.any(nxt != cum)

        cum, _ = jax.lax.while_loop(
            fp_cond, fp_body, (alive0, jnp.bool_(True)))
        keep_ref[0:1, pl.ds(off, _T)] = cum

        def chunk_step(k, c2):
            cb = k * _W
            bx1 = cols_ref[0:1, pl.ds(cb, _W)]
            by1 = cols_ref[1:2, pl.ds(cb, _W)]
            bx2 = cols_ref[2:3, pl.ds(cb, _W)]
            by2 = cols_ref[3:4, pl.ds(cb, _W)]
            bar = cols_ref[4:5, pl.ds(cb, _W)]
            oc = _overlap(tx1, ty1, tx2, ty2, tar, bx1, by1, bx2, by2, bar)
            scf = jnp.where(oc, jnp.float32(1.0), jnp.float32(0.0))
            hits = jnp.dot(cum, scf, preferred_element_type=jnp.float32)
            colid = cb + jax.lax.broadcasted_iota(jnp.int32, (1, _W), 1)
            kill = (hits > 0) & (colid >= off + _T)
            cur = keep_ref[0:1, pl.ds(cb, _W)]
            keep_ref[0:1, pl.ds(cb, _W)] = jnp.where(
                kill, jnp.float32(0.0), cur)
            return c2

        kmin = (off + _T) // _W
        jax.lax.fori_loop(kmin, _NC, chunk_step, 0)
        return carry

    jax.lax.fori_loop(0, _NT, tile_step, 0)
    keep_ref[...] = keep_ref[...] * cols_ref[5:6, :]


def _run_nms(cols, interpret=False):
    return pl.pallas_call(
        _nms_body,
        out_shape=jax.ShapeDtypeStruct((1, _NPAD), jnp.float32),
        interpret=interpret,
    )(cols)


def kernel(boxes, scores):
    n = boxes.shape[0]
    idx = jnp.arange(n, dtype=jnp.int32)
    sneg, sx1, sy1, sx2, sy2, sidx = jax.lax.sort(
        (-scores, boxes[:, 0], boxes[:, 1], boxes[:, 2], boxes[:, 3], idx),
        num_keys=1, is_stable=True)
    ssc = -sneg
    area = jnp.maximum(sx2 - sx1, 0.0) * jnp.maximum(sy2 - sy1, 0.0)
    feats = jnp.stack([sx1, sy1, sx2, sy2, area, ssc], axis=0)
    cols = jnp.zeros((8, _NPAD), jnp.float32).at[:6, :n].set(feats)
    masked_sorted = _run_nms(cols)[0, :n]
    _, out = jax.lax.sort((sidx, masked_sorted), num_keys=1)
    return out
